# baseline (speedup 1.0000x reference)
import functools

import jax
import jax.numpy as jnp
import numpy as np
from jax import lax
from jax.experimental import pallas as pl
from jax.experimental.pallas import tpu as pltpu
from jax.experimental.pallas import tpu_sc as plsc

ROI_LO_I, ROI_HI_I = 40, 400
M_DIPS = 6
MIN_AREA = 1e-05
W_AREA = 1.0
W_CENTROID = 1.0
W_DEPTH = 0.2
UNDERFILL_FACTOR = 2.0
B, L = 64, 2048
HALF = 10
WN = 2 * HALF + 1
NEG = float("-inf")

NSUB = 16
NCORE = 2
NW = NCORE * NSUB
ROWS_PER = B // NW


def _log_taps():
    sigma = 2.0
    radius = int(max(1.0, 3.0 * sigma))
    x = np.arange(-radius, radius + 1, dtype=np.float32)
    s2 = np.float32(sigma * sigma)
    g = np.exp(-(x ** 2) / (2.0 * s2)).astype(np.float32)
    taps = ((x ** 2 - s2) / s2 ** 2 * g).astype(np.float32)
    taps = (taps - taps.mean()).astype(np.float32)
    return taps


_TAPS = _log_taps()

BK0 = 32
BW = 512


def _detect_body(t_ref, centers_ref, pool_ref):
    acc = jnp.zeros((B, BW), jnp.float32)
    for k in range(_TAPS.shape[0]):
        off = BK0 - 6 + k
        acc = acc + float(_TAPS[k]) * t_ref[:, off:off + BW]
    colb = lax.broadcasted_iota(jnp.int32, (B, BW), 1)
    roi = ((colb >= ROI_LO_I - BK0) & (colb <= ROI_HI_I - BK0)).astype(
        jnp.float32)
    scores = -acc * roi

    pool_ref[:, 0:64] = jnp.zeros((B, 64), jnp.float32)
    pool_ref[:, 64 + BW:64 + BW + 64] = jnp.zeros((B, 64), jnp.float32)
    pool_ref[:, 64:64 + BW] = scores
    pooled = pool_ref[:, 59:59 + BW]
    for d in range(1, 11):
        pooled = jnp.maximum(pooled, pool_ref[:, 59 + d:59 + d + BW])

    mean = jnp.sum(scores, axis=1, keepdims=True) * (1.0 / L)
    keep = (scores == pooled) & (scores > mean)
    masked = jnp.where(keep & (scores > 0.0), scores, NEG)

    cbs = []
    pvs = []
    for tk in range(M_DIPS):
        m = jnp.max(masked, axis=1, keepdims=True)
        ismax = masked == m
        cb = jnp.min(jnp.where(ismax, colb, BW), axis=1, keepdims=True)
        cbs.append(cb)
        pvs.append(m > NEG)
        masked = jnp.where(colb == cb, NEG, masked)

    p_cnt = jnp.zeros((B, 1), jnp.int32)
    for pv in pvs:
        p_cnt = p_cnt + pv.astype(jnp.int32)
    neg_mean = mean < 0.0

    col16 = lax.broadcasted_iota(jnp.int32, (B, 16), 1)
    centers16 = jnp.full((B, 16), -1, jnp.int32)
    for tk in range(M_DIPS):
        c = jnp.where(pvs[tk], BK0 + cbs[tk], tk - p_cnt)
        enc = jnp.where(pvs[tk] | neg_mean, c, -1)
        centers16 = jnp.where(col16 == tk, enc, centers16)
    centers_ref[:] = centers16


def _detect(target):
    return pl.pallas_call(
        _detect_body,
        out_shape=jax.ShapeDtypeStruct((B, 16), jnp.int32),
        scratch_shapes=[
            pltpu.VMEM((B, BW + 128), jnp.float32),
        ],
    )(target)


def _sc_gather_body(tgt_hbm, pred_hbm, cenc_hbm, tv_hbm, pv_hbm,
                    trows, prows, crows, twin, pwin, sem):
    wid = lax.axis_index("s") * NCORE + lax.axis_index("c")
    base = wid * ROWS_PER
    copies = [
        pltpu.async_copy(cenc_hbm.at[pl.ds(base, ROWS_PER)], crows, sem),
        pltpu.async_copy(tgt_hbm.at[pl.ds(base, ROWS_PER)], trows, sem),
        pltpu.async_copy(pred_hbm.at[pl.ds(base, ROWS_PER)], prows, sem),
    ]
    for cp in copies:
        cp.wait()
    for r in range(ROWS_PER):
        ce = crows[r]
        c = jnp.maximum(ce, 0)
        s = jnp.maximum(c - HALF, 0)
        e = jnp.minimum(c + HALF, L - 1)
        rvec = jnp.full((16,), r, jnp.int32)

        def jbody(j, carry, r=r, s=s, e=e, rvec=rvec):
            idx = jnp.minimum(s + j, e)
            twin[j, r] = plsc.load_gather(trows, [rvec, idx])
            pwin[j, r] = plsc.load_gather(prows, [rvec, idx])
            return carry

        lax.fori_loop(0, WN, jbody, 0)
    outs = [
        pltpu.async_copy(twin, tv_hbm.at[:, pl.ds(base, ROWS_PER)], sem),
        pltpu.async_copy(pwin, pv_hbm.at[:, pl.ds(base, ROWS_PER)], sem),
    ]
    for cp in outs:
        cp.wait()


@functools.cache
def _sc_gather():
  return pl.kernel(
    _sc_gather_body,
    out_type=[
        jax.ShapeDtypeStruct((WN, B, 16), jnp.float32),
        jax.ShapeDtypeStruct((WN, B, 16), jnp.float32),
    ],
    mesh=plsc.VectorSubcoreMesh(core_axis_name="c", subcore_axis_name="s",
                                num_cores=NCORE, num_subcores=NSUB),
    compiler_params=pltpu.CompilerParams(needs_layout_passes=False),
    scratch_types=[
        pltpu.VMEM((ROWS_PER, L), jnp.float32),
        pltpu.VMEM((ROWS_PER, L), jnp.float32),
        pltpu.VMEM((ROWS_PER, 16), jnp.int32),
        pltpu.VMEM((WN, ROWS_PER, 16), jnp.float32),
        pltpu.VMEM((WN, ROWS_PER, 16), jnp.float32),
        pltpu.SemaphoreType.DMA,
    ],
  )


def _terms_body(tv_ref, pv_ref, c8_ref, out_ref, tts_ref, dts_ref, dps_ref):
    ce = c8_ref[:]
    vld_b = ce >= 0
    c = jnp.where(vld_b, ce, 0)
    s = jnp.maximum(c - HALF, 0)
    e = jnp.minimum(c + HALF, L - 1)
    n = e - s
    nf = n.astype(jnp.float32)
    lam_s = 300.0 + 0.5 * s.astype(jnp.float32)
    lam_e = 300.0 + 0.5 * e.astype(jnp.float32)
    dlam = lam_e - lam_s + 1e-6
    ts = tv_ref[0]
    te = tv_ref[WN - 1]
    ps = pv_ref[0]
    pe = pv_ref[WN - 1]
    def _lseg(j):
        idx = jnp.minimum(s + j, e)
        return 300.0 + 0.5 * idx.astype(jnp.float32)

    for j in range(WN):
        tts_ref[j] = (_lseg(j) - lam_s) / dlam
    for j in range(WN):
        tt = tts_ref[j]
        cont_t = jnp.maximum((1.0 - tt) * ts + tt * te, 1e-6)
        cont_p = jnp.maximum((1.0 - tt) * ps + tt * pe, 1e-6)
        dts_ref[j] = jnp.clip(1.0 - jnp.clip(tv_ref[j] / cont_t, 0.0, 2.0),
                              0.0, 1.0)
        dps_ref[j] = jnp.clip(1.0 - jnp.clip(pv_ref[j] / cont_p, 0.0, 2.0),
                              0.0, 1.0)
    zero = jnp.zeros((B, 16), jnp.float32)
    area_t = zero
    area_p = zero
    ct_num = zero
    ct_den = zero
    cp_num = zero
    cp_den = zero
    dsum = zero
    prev_dt = zero
    prev_dp = zero
    prev_lseg = zero
    for j in range(WN):
        lseg = _lseg(j)
        dt = dts_ref[j]
        dp = dps_ref[j]
        jf = float(j)
        pm = jf <= nf
        if j > 0:
            sm = (jf - 1.0) < nf
            dl = lseg - prev_lseg
            area_t = area_t + jnp.where(sm, (dt + prev_dt) * 0.5 * dl, 0.0)
            area_p = area_p + jnp.where(sm, (dp + prev_dp) * 0.5 * dl, 0.0)
        wt = dt + 1e-7
        wp = dp + 1e-7
        ct_num = ct_num + jnp.where(pm, lseg * wt, 0.0)
        ct_den = ct_den + jnp.where(pm, wt, 0.0)
        cp_num = cp_num + jnp.where(pm, lseg * wp, 0.0)
        cp_den = cp_den + jnp.where(pm, wp, 0.0)
        dsum = dsum + jnp.where(pm, jnp.abs(dp - dt), 0.0)
        prev_dt = dt
        prev_dp = dp
        prev_lseg = lseg
    rel_err = jnp.abs(area_p - area_t) / (area_t + 1e-7)
    underfill = jnp.maximum(area_t - area_p, 0.0) / (area_t + 1e-7)
    area_term = rel_err + (UNDERFILL_FACTOR - 1.0) * underfill
    centroid_term = jnp.abs(cp_num / cp_den - ct_num / ct_den)
    depth_term = dsum / (nf + 1.0)
    valid = vld_b & (e > s) & jnp.logical_not(area_t < MIN_AREA)
    cnt = jnp.sum(jnp.where(valid, 1.0, 0.0))
    a = jnp.sum(jnp.where(valid, area_term, 0.0))
    cc = jnp.sum(jnp.where(valid, centroid_term, 0.0))
    dd = jnp.sum(jnp.where(valid, depth_term, 0.0))
    den = jnp.maximum(cnt, 1.0)
    num = W_AREA * a + W_CENTROID * cc + W_DEPTH * dd
    loss = jnp.full((1, 1), num) / jnp.full((1, 1), den)
    loss = jnp.where(jnp.full((1, 1), cnt) > 0.0, loss,
                     jnp.zeros((1, 1), jnp.float32))
    out_ref[:] = loss


def _terms(tv, pv, c8):
    return pl.pallas_call(
        _terms_body,
        out_shape=jax.ShapeDtypeStruct((1, 1), jnp.float32),
        scratch_shapes=[
            pltpu.VMEM((WN, B, 16), jnp.float32),
            pltpu.VMEM((WN, B, 16), jnp.float32),
            pltpu.VMEM((WN, B, 16), jnp.float32),
        ],
    )(tv, pv, c8)


def kernel(prediction, target, lam_nm):
    del lam_nm
    pred = prediction.astype(jnp.float32)
    tgt = target.astype(jnp.float32)
    centers_enc = _detect(tgt)
    tv, pv = _sc_gather()(tgt, pred, centers_enc)
    loss = _terms(tv, pv, centers_enc)
    return loss.reshape(())
